# Initial kernel scaffold; baseline (speedup 1.0000x reference)
import functools

import jax
import jax.numpy as jnp
from jax import lax
from jax.experimental import pallas as pl
from jax.experimental.pallas import tpu as pltpu
from jax.experimental.pallas import tpu_sc as plsc

N = 10000
E = 320000
D = 128
H = 128
OUT = 40

NC, NS = 2, 16
NTILE = NC * NS
CHUNK = 128
EPAD = ((E + NTILE * CHUNK - 1) // (NTILE * CHUNK)) * NTILE * CHUNK
EPT = EPAD // NTILE
NCHUNK = EPT // CHUNK
NPAD = 10240
RPT = NPAD // NS
DEGW = 16

BR = 1280
GRID = NPAD // BR

_mesh = plsc.VectorSubcoreMesh(
    core_axis_name="c", subcore_axis_name="s", num_cores=NC, num_subcores=NS
)


@functools.partial(
    pl.kernel,
    out_type=jax.ShapeDtypeStruct((NC, NPAD, DEGW), jnp.float32),
    mesh=_mesh,
    scratch_types=[
        pltpu.VMEM((CHUNK,), jnp.int32),
        pltpu.VMEM((CHUNK, DEGW), jnp.float32),
        pltpu.VMEM_SHARED((NPAD, DEGW), jnp.float32),
    ],
)
def _sc_degree(dst_hbm, ones_hbm, zeros_hbm, deg_hbm, idx_d, ones_v, hist):
    c = lax.axis_index("c")
    s = lax.axis_index("s")
    r0 = s * RPT
    pltpu.sync_copy(zeros_hbm.at[pl.ds(r0, RPT)], hist.at[pl.ds(r0, RPT)])
    pltpu.sync_copy(ones_hbm, ones_v)
    plsc.subcore_barrier()
    base = c * (EPAD // NC) + s * EPT

    def step(i, carry):
        b = base + i * CHUNK
        pltpu.sync_copy(dst_hbm.at[pl.ds(b, CHUNK)], idx_d)
        pltpu.sync_copy(ones_v, hist.at[idx_d], add=True)
        return carry

    lax.fori_loop(0, NCHUNK, step, 0)
    plsc.subcore_barrier()
    pltpu.sync_copy(hist.at[pl.ds(r0, RPT)], deg_hbm.at[c, pl.ds(r0, RPT)])


@functools.partial(
    pl.kernel,
    out_type=jax.ShapeDtypeStruct((NC, NPAD, H), jnp.float32),
    mesh=_mesh,
    scratch_types=[
        pltpu.VMEM((CHUNK,), jnp.int32),
        pltpu.VMEM((CHUNK,), jnp.int32),
        pltpu.VMEM((CHUNK, H), jnp.float32),
        pltpu.SemaphoreType.DMA,
        pltpu.VMEM_SHARED((NPAD, H), jnp.float32),
    ],
)
def _sc_aggregate(zs_hbm, src_hbm, dst_hbm, agg_hbm, idx_s, idx_d, rows, sem, acc):
    c = lax.axis_index("c")
    s = lax.axis_index("s")
    r0 = s * RPT
    pltpu.sync_copy(zs_hbm.at[pl.ds(r0, RPT)], acc.at[pl.ds(r0, RPT)])
    plsc.subcore_barrier()
    base = c * (EPAD // NC) + s * EPT

    def step(i, carry):
        b = base + i * CHUNK
        pltpu.sync_copy(src_hbm.at[pl.ds(b, CHUNK)], idx_s)
        pltpu.async_copy(zs_hbm.at[idx_s], rows, sem).wait()
        pltpu.sync_copy(dst_hbm.at[pl.ds(b, CHUNK)], idx_d)
        pltpu.sync_copy(rows, acc.at[idx_d], add=True)
        return carry

    lax.fori_loop(0, NCHUNK, step, 0)
    plsc.subcore_barrier()
    pltpu.sync_copy(acc.at[pl.ds(r0, RPT)], agg_hbm.at[c, pl.ds(r0, RPT)])


def _row_spec():
    return pl.BlockSpec((BR, H), lambda i: (i, 0))


def _full_spec(shape):
    return pl.BlockSpec(shape, lambda i: tuple(0 for _ in shape))


def _layer1_body(x_ref, d0_ref, d1_ref, w_ref, zs_ref, dinv_ref):
    i = pl.program_id(0)
    deg = d0_ref[:, 0:1] + d1_ref[:, 0:1] + 1.0
    dinv = jnp.broadcast_to(lax.rsqrt(deg), (BR, H))
    rid = lax.broadcasted_iota(jnp.int32, (BR, H), 0) + i * BR
    dinv = jnp.where(rid < N, dinv, 0.0)
    dinv_ref[...] = dinv
    zs_ref[...] = jnp.dot(dinv * x_ref[...], w_ref[...],
                          preferred_element_type=jnp.float32)


def _tc_layer1(xp, d0, d1, w):
    return pl.pallas_call(
        _layer1_body,
        grid=(GRID,),
        in_specs=[
            _row_spec(),
            pl.BlockSpec((BR, DEGW), lambda i: (i, 0)),
            pl.BlockSpec((BR, DEGW), lambda i: (i, 0)),
            _full_spec((D, H)),
        ],
        out_specs=[_row_spec(), _row_spec()],
        out_shape=[
            jax.ShapeDtypeStruct((NPAD, H), jnp.float32),
            jax.ShapeDtypeStruct((NPAD, H), jnp.float32),
        ],
    )(xp, d0, d1, w)


def _combine_body(a0_ref, a1_ref, zs_ref, dinv_ref, b_ref, w_ref, out_ref):
    dinv = dinv_ref[...]
    h = jnp.maximum(dinv * (a0_ref[...] + a1_ref[...] - zs_ref[...]) + b_ref[...], 0.0)
    out_ref[...] = jnp.dot(dinv * h, w_ref[...], preferred_element_type=jnp.float32)


def _tc_combine(a0, a1, zs, dinv2d, b, w):
    return pl.pallas_call(
        _combine_body,
        grid=(GRID,),
        in_specs=[
            _row_spec(), _row_spec(), _row_spec(), _row_spec(),
            _full_spec((1, H)), _full_spec((H, H)),
        ],
        out_specs=_row_spec(),
        out_shape=jax.ShapeDtypeStruct((NPAD, H), jnp.float32),
    )(a0, a1, zs, dinv2d, b, w)


def _final_body(a0_ref, a1_ref, zs_ref, dinv_ref, b_ref, wr_ref, br_ref, out_ref):
    dinv = dinv_ref[...]
    h = jnp.maximum(dinv * (a0_ref[...] + a1_ref[...] - zs_ref[...]) + b_ref[...], 0.0)
    logits = jnp.dot(h, wr_ref[...], preferred_element_type=jnp.float32) + br_ref[...]
    m = jnp.max(logits, axis=1, keepdims=True)
    e = jnp.exp(logits - m)
    out_ref[...] = e / jnp.sum(e, axis=1, keepdims=True)


def _tc_final(a0, a1, zs, dinv2d, b, wr_pad, br_pad):
    return pl.pallas_call(
        _final_body,
        grid=(GRID,),
        in_specs=[
            _row_spec(), _row_spec(), _row_spec(), _row_spec(),
            _full_spec((1, H)), _full_spec((H, H)), _full_spec((1, H)),
        ],
        out_specs=_row_spec(),
        out_shape=jax.ShapeDtypeStruct((NPAD, H), jnp.float32),
    )(a0, a1, zs, dinv2d, b, wr_pad, br_pad)


def kernel(x, edge_index, W1, b1, W2, b2, W3, b3, Wr, br):
    f32 = jnp.float32
    xp = jnp.zeros((NPAD, D), f32).at[:N].set(x)
    src = jnp.full((EPAD,), N, jnp.int32).at[:E].set(edge_index[0])
    dst = jnp.full((EPAD,), N, jnp.int32).at[:E].set(edge_index[1])
    ones_rows = jnp.ones((CHUNK, DEGW), f32)
    zeros_hist = jnp.zeros((NPAD, DEGW), f32)
    wr_pad = jnp.zeros((H, H), f32).at[:, :OUT].set(Wr)
    br_pad = jnp.full((1, H), -1e30, f32).at[0, :OUT].set(br)

    d2 = _sc_degree(dst, ones_rows, zeros_hist)
    zs1, dinv2d = _tc_layer1(xp, d2[0], d2[1], W1)
    a1 = _sc_aggregate(zs1, src, dst)
    zs2 = _tc_combine(a1[0], a1[1], zs1, dinv2d, b1.reshape(1, H), W2)
    a2 = _sc_aggregate(zs2, src, dst)
    zs3 = _tc_combine(a2[0], a2[1], zs2, dinv2d, b2.reshape(1, H), W3)
    a3 = _sc_aggregate(zs3, src, dst)
    probs = _tc_final(a3[0], a3[1], zs3, dinv2d, b3.reshape(1, H), wr_pad, br_pad)
    return probs[:N, :OUT]


# speedup vs baseline: 8.6362x; 8.6362x over previous
import functools

import jax
import jax.numpy as jnp
from jax import lax
from jax.experimental import pallas as pl
from jax.experimental.pallas import tpu as pltpu
from jax.experimental.pallas import tpu_sc as plsc

N = 10000
E = 320000
D = 128
H = 128
OUT = 40

NC, NS = 2, 16
NTILE = NC * NS
CHUNK = 128
EPAD = ((E + NTILE * CHUNK - 1) // (NTILE * CHUNK)) * NTILE * CHUNK
EPT = EPAD // NTILE
NCHUNK = EPT // CHUNK
NPAD = 10240
RPT = NPAD // NS
DEGW = 128

BR = 1280
GRID = NPAD // BR

_mesh = plsc.VectorSubcoreMesh(
    core_axis_name="c", subcore_axis_name="s", num_cores=NC, num_subcores=NS
)


@functools.partial(
    pl.kernel,
    out_type=jax.ShapeDtypeStruct((NC, NPAD, DEGW), jnp.float32),
    mesh=_mesh,
    scratch_types=[
        pltpu.VMEM((CHUNK,), jnp.int32),
        pltpu.VMEM((CHUNK, DEGW), jnp.float32),
        pltpu.VMEM_SHARED((NPAD, DEGW), jnp.float32),
    ],
)
def _sc_degree(dst_hbm, ones_hbm, zeros_hbm, deg_hbm, idx_d, ones_v, hist):
    c = lax.axis_index("c")
    s = lax.axis_index("s")
    r0 = s * RPT
    pltpu.sync_copy(zeros_hbm.at[pl.ds(r0, RPT)], hist.at[pl.ds(r0, RPT)])
    pltpu.sync_copy(ones_hbm, ones_v)
    plsc.subcore_barrier()
    base = c * (EPAD // NC) + s * EPT

    def step(i, carry):
        b = base + i * CHUNK
        pltpu.sync_copy(dst_hbm.at[pl.ds(b, CHUNK)], idx_d)
        pltpu.sync_copy(ones_v, hist.at[idx_d], add=True)
        return carry

    lax.fori_loop(0, NCHUNK, step, 0)
    plsc.subcore_barrier()
    pltpu.sync_copy(hist.at[pl.ds(r0, RPT)], deg_hbm.at[c, pl.ds(r0, RPT)])


@functools.partial(
    pl.kernel,
    out_type=jax.ShapeDtypeStruct((NC, NPAD, H), jnp.float32),
    mesh=_mesh,
    scratch_types=[
        pltpu.VMEM((CHUNK,), jnp.int32),
        pltpu.VMEM((CHUNK,), jnp.int32),
        pltpu.VMEM((CHUNK, H), jnp.float32),
        pltpu.SemaphoreType.DMA,
        pltpu.VMEM_SHARED((NPAD, H), jnp.float32),
    ],
)
def _sc_aggregate(zs_hbm, src_hbm, dst_hbm, agg_hbm, idx_s, idx_d, rows, sem, acc):
    c = lax.axis_index("c")
    s = lax.axis_index("s")
    r0 = s * RPT
    pltpu.sync_copy(zs_hbm.at[pl.ds(r0, RPT)], acc.at[pl.ds(r0, RPT)])
    plsc.subcore_barrier()
    base = c * (EPAD // NC) + s * EPT

    def step(i, carry):
        b = base + i * CHUNK
        pltpu.sync_copy(src_hbm.at[pl.ds(b, CHUNK)], idx_s)
        pltpu.async_copy(zs_hbm.at[idx_s], rows, sem).wait()
        pltpu.sync_copy(dst_hbm.at[pl.ds(b, CHUNK)], idx_d)
        pltpu.sync_copy(rows, acc.at[idx_d], add=True)
        return carry

    lax.fori_loop(0, NCHUNK, step, 0)
    plsc.subcore_barrier()
    pltpu.sync_copy(acc.at[pl.ds(r0, RPT)], agg_hbm.at[c, pl.ds(r0, RPT)])


def _row_spec():
    return pl.BlockSpec((BR, H), lambda i: (i, 0))


def _full_spec(shape):
    return pl.BlockSpec(shape, lambda i: tuple(0 for _ in shape))


def _layer1_body(x_ref, d0_ref, d1_ref, w_ref, zs_ref, dinv_ref):
    i = pl.program_id(0)
    deg = d0_ref[:, 0:1] + d1_ref[:, 0:1] + 1.0
    dinv = jnp.broadcast_to(lax.rsqrt(deg), (BR, H))
    rid = lax.broadcasted_iota(jnp.int32, (BR, H), 0) + i * BR
    dinv = jnp.where(rid < N, dinv, 0.0)
    dinv_ref[...] = dinv
    zs_ref[...] = jnp.dot(dinv * x_ref[...], w_ref[...],
                          preferred_element_type=jnp.float32)


def _tc_layer1(xp, d0, d1, w):
    return pl.pallas_call(
        _layer1_body,
        grid=(GRID,),
        in_specs=[
            _row_spec(),
            pl.BlockSpec((BR, DEGW), lambda i: (i, 0)),
            pl.BlockSpec((BR, DEGW), lambda i: (i, 0)),
            _full_spec((D, H)),
        ],
        out_specs=[_row_spec(), _row_spec()],
        out_shape=[
            jax.ShapeDtypeStruct((NPAD, H), jnp.float32),
            jax.ShapeDtypeStruct((NPAD, H), jnp.float32),
        ],
    )(xp, d0, d1, w)


def _combine_body(a0_ref, a1_ref, zs_ref, dinv_ref, b_ref, w_ref, out_ref):
    dinv = dinv_ref[...]
    h = jnp.maximum(dinv * (a0_ref[...] + a1_ref[...] - zs_ref[...]) + b_ref[...], 0.0)
    out_ref[...] = jnp.dot(dinv * h, w_ref[...], preferred_element_type=jnp.float32)


def _tc_combine(a0, a1, zs, dinv2d, b, w):
    return pl.pallas_call(
        _combine_body,
        grid=(GRID,),
        in_specs=[
            _row_spec(), _row_spec(), _row_spec(), _row_spec(),
            _full_spec((1, H)), _full_spec((H, H)),
        ],
        out_specs=_row_spec(),
        out_shape=jax.ShapeDtypeStruct((NPAD, H), jnp.float32),
    )(a0, a1, zs, dinv2d, b, w)


def _final_body(a0_ref, a1_ref, zs_ref, dinv_ref, b_ref, wr_ref, br_ref, out_ref):
    dinv = dinv_ref[...]
    h = jnp.maximum(dinv * (a0_ref[...] + a1_ref[...] - zs_ref[...]) + b_ref[...], 0.0)
    logits = jnp.dot(h, wr_ref[...], preferred_element_type=jnp.float32) + br_ref[...]
    m = jnp.max(logits, axis=1, keepdims=True)
    e = jnp.exp(logits - m)
    out_ref[...] = e / jnp.sum(e, axis=1, keepdims=True)


def _tc_final(a0, a1, zs, dinv2d, b, wr_pad, br_pad):
    return pl.pallas_call(
        _final_body,
        grid=(GRID,),
        in_specs=[
            _row_spec(), _row_spec(), _row_spec(), _row_spec(),
            _full_spec((1, H)), _full_spec((H, H)), _full_spec((1, H)),
        ],
        out_specs=_row_spec(),
        out_shape=jax.ShapeDtypeStruct((NPAD, H), jnp.float32),
    )(a0, a1, zs, dinv2d, b, wr_pad, br_pad)


def kernel(x, edge_index, W1, b1, W2, b2, W3, b3, Wr, br):
    f32 = jnp.float32
    xp = jnp.zeros((NPAD, D), f32).at[:N].set(x)
    src = jnp.full((EPAD,), N, jnp.int32).at[:E].set(edge_index[0])
    dst = jnp.full((EPAD,), N, jnp.int32).at[:E].set(edge_index[1])
    ones_rows = jnp.ones((CHUNK, DEGW), f32)
    zeros_hist = jnp.zeros((NPAD, DEGW), f32)
    wr_pad = jnp.zeros((H, H), f32).at[:, :OUT].set(Wr)
    br_pad = jnp.full((1, H), -1e30, f32).at[0, :OUT].set(br)

    d2 = _sc_degree(dst, ones_rows, zeros_hist)
    zs1, dinv2d = _tc_layer1(xp, d2[0], d2[1], W1)
    a1 = _sc_aggregate(zs1, src, dst)
    zs2 = _tc_combine(a1[0], a1[1], zs1, dinv2d, b1.reshape(1, H), W2)
    a2 = _sc_aggregate(zs2, src, dst)
    zs3 = _tc_combine(a2[0], a2[1], zs2, dinv2d, b2.reshape(1, H), W3)
    a3 = _sc_aggregate(zs3, src, dst)
    probs = _tc_final(a3[0], a3[1], zs3, dinv2d, b3.reshape(1, H), wr_pad, br_pad)
    return probs[:N, :OUT]
